# baseline (speedup 1.0000x reference)
import functools

import jax
import jax.numpy as jnp
from jax import lax
from jax.experimental import pallas as pl
from jax.experimental.pallas import tpu as pltpu
from jax.experimental.pallas import tpu_sc as plsc

EMB_D = 128
IDX_CHUNK = 128


def _build_gather(batch: int, vocab: int, d: int):
    info = plsc.get_sparse_core_info()
    nw = info.num_cores * info.num_subcores
    b_per_w = batch // nw
    n_chunks = b_per_w // IDX_CHUNK
    mesh = plsc.VectorSubcoreMesh(core_axis_name="c", subcore_axis_name="s")

    @functools.partial(
        pl.kernel,
        mesh=mesh,
        out_type=jax.ShapeDtypeStruct((batch, d), jnp.float32),
        scratch_types=[
            pltpu.VMEM((n_chunks, IDX_CHUNK), jnp.int32),
            pltpu.VMEM((b_per_w, d), jnp.float32),
        ]
        + [pltpu.SemaphoreType.DMA] * (n_chunks + 1),
    )
    def gather_kernel(idx_hbm, table_hbm, out_hbm, idx_v, rows_v, *sems):
        gsems, ssem = sems[:n_chunks], sems[n_chunks]
        wid = lax.axis_index("s") * info.num_cores + lax.axis_index("c")
        base = wid * b_per_w
        pltpu.sync_copy(idx_hbm.at[wid], idx_v)
        copies = []
        for j in range(n_chunks):
            copies.append(
                pltpu.async_copy(
                    table_hbm.at[idx_v.at[j]],
                    rows_v.at[pl.ds(j * IDX_CHUNK, IDX_CHUNK)],
                    gsems[j],
                )
            )
        stores = []
        for j in range(n_chunks):
            copies[j].wait()
            stores.append(
                pltpu.async_copy(
                    rows_v.at[pl.ds(j * IDX_CHUNK, IDX_CHUNK)],
                    out_hbm.at[pl.ds(base + j * IDX_CHUNK, IDX_CHUNK)],
                    ssem,
                )
            )
        for s in stores:
            s.wait()

    return gather_kernel, nw, n_chunks


def kernel(x, atom_emb):
    batch = x.shape[0]
    vocab, d = atom_emb.shape
    gather_kernel, nw, n_chunks = _build_gather(batch, vocab, d)
    idx = x.astype(jnp.int32).reshape(nw, n_chunks, IDX_CHUNK)
    return gather_kernel(idx, atom_emb)


# speedup vs baseline: 1.0278x; 1.0278x over previous
import functools

import jax
import jax.numpy as jnp
from jax import lax
from jax.experimental import pallas as pl
from jax.experimental.pallas import tpu as pltpu
from jax.experimental.pallas import tpu_sc as plsc

IDX_CHUNK = 128


def _build_gather(batch: int, d: int):
    info = plsc.get_sparse_core_info()
    nw = info.num_cores * info.num_subcores
    b_per_w = batch // nw
    n_chunks = b_per_w // IDX_CHUNK
    mesh = plsc.VectorSubcoreMesh(core_axis_name="c", subcore_axis_name="s")

    @functools.partial(
        pl.kernel,
        mesh=mesh,
        out_type=jax.ShapeDtypeStruct((batch, d), jnp.float32),
        scratch_types=[
            pltpu.VMEM((b_per_w,), jnp.int32),
            pltpu.VMEM((b_per_w, d), jnp.float32),
            pltpu.SemaphoreType.DMA,
        ],
    )
    def gather_kernel(idx_hbm, table_hbm, out_hbm, idx_v, rows_v, sem):
        wid = lax.axis_index("s") * info.num_cores + lax.axis_index("c")
        base = wid * b_per_w
        pltpu.sync_copy(idx_hbm.at[pl.ds(base, b_per_w)], idx_v)
        copies = []
        for j in range(n_chunks):
            copies.append(
                pltpu.async_copy(
                    table_hbm.at[idx_v.at[pl.ds(j * IDX_CHUNK, IDX_CHUNK)]],
                    rows_v.at[pl.ds(j * IDX_CHUNK, IDX_CHUNK)],
                    sem,
                )
            )
        for c in copies:
            c.wait()
        pltpu.sync_copy(rows_v, out_hbm.at[pl.ds(base, b_per_w)])

    return gather_kernel


def kernel(x, atom_emb):
    batch = x.shape[0]
    d = atom_emb.shape[1]
    gather_kernel = _build_gather(batch, d)
    return gather_kernel(x.astype(jnp.int32), atom_emb)


# speedup vs baseline: 1.1639x; 1.1325x over previous
import functools

import jax
import jax.numpy as jnp
from jax import lax
from jax.experimental import pallas as pl
from jax.experimental.pallas import tpu as pltpu
from jax.experimental.pallas import tpu_sc as plsc

IDX_CHUNK = 128


def _build_gather(batch: int, vocab: int, d: int):
    info = plsc.get_sparse_core_info()
    nw = info.num_cores * info.num_subcores
    b_per_w = batch // nw
    n_chunks = b_per_w // IDX_CHUNK
    mesh = plsc.VectorSubcoreMesh(core_axis_name="c", subcore_axis_name="s")

    @functools.partial(
        pl.kernel,
        mesh=mesh,
        out_type=jax.ShapeDtypeStruct((batch, d), jnp.float32),
        scratch_types=[
            pltpu.VMEM((b_per_w,), jnp.int32),
            pltpu.VMEM((b_per_w, d), jnp.float32),
            pltpu.VMEM_SHARED((vocab, d), jnp.float32),
        ]
        + [pltpu.SemaphoreType.DMA] * (n_chunks + 1),
    )
    def gather_kernel(idx_hbm, table_hbm, out_hbm, idx_v, rows_v, table_sh, *sems):
        gsems, ssem = sems[:n_chunks], sems[n_chunks]
        cid = lax.axis_index("c")
        sid = lax.axis_index("s")
        wid = sid * info.num_cores + cid
        base = wid * b_per_w

        @pl.when(sid == 0)
        def _():
            pltpu.sync_copy(table_hbm, table_sh)

        pltpu.sync_copy(idx_hbm.at[pl.ds(base, b_per_w)], idx_v)
        plsc.subcore_barrier()

        copies = []
        for j in range(n_chunks):
            copies.append(
                pltpu.async_copy(
                    table_sh.at[idx_v.at[pl.ds(j * IDX_CHUNK, IDX_CHUNK)]],
                    rows_v.at[pl.ds(j * IDX_CHUNK, IDX_CHUNK)],
                    gsems[j],
                )
            )
        stores = []
        for j in range(n_chunks):
            copies[j].wait()
            stores.append(
                pltpu.async_copy(
                    rows_v.at[pl.ds(j * IDX_CHUNK, IDX_CHUNK)],
                    out_hbm.at[pl.ds(base + j * IDX_CHUNK, IDX_CHUNK)],
                    ssem,
                )
            )
        for s in stores:
            s.wait()

    return gather_kernel


def kernel(x, atom_emb):
    batch = x.shape[0]
    vocab, d = atom_emb.shape
    gather_kernel = _build_gather(batch, vocab, d)
    return gather_kernel(x.astype(jnp.int32), atom_emb)
